# baseline (speedup 1.0000x reference)
import functools

import jax
import jax.numpy as jnp
from jax import lax
from jax.experimental import pallas as pl
from jax.experimental.pallas import tpu as pltpu
from jax.experimental.pallas import tpu_sc as plsc

_IOU_THR = 0.6
_MAX_OUT = 100
_NS = 16
_PER = 1280
_NP = _NS * _PER
_VPT = _PER // 16
_OUT_PAD = 112
_BIG = 1 << 30


def _dyn_lane(v, lane):
    acc = v[0]
    for l in range(1, 16):
        acc = jnp.where(lane == l, v[l], acc)
    return acc


def _sc_body(n_real, num_classes,
             bt_hbm, st_hbm, out_f, out_i,
             scbuf, boxbuf, smbuf, lbbuf, areabuf,
             stagef, stagei, valbuf, idxbuf, winfbuf, winibuf,
             outfb, outib, s0buf,
             vals_sh, idx_sh, winf_sh, wini_sh):
    sid = lax.axis_index("s")
    base = sid * _PER
    l16 = lax.broadcasted_iota(jnp.int32, (16,), 0)

    nbox = 4 * _PER
    nsc = num_classes * _PER
    pltpu.sync_copy(bt_hbm.at[pl.ds(sid * nbox, nbox)], boxbuf)
    pltpu.sync_copy(st_hbm.at[pl.ds(sid * nsc, nsc)], scbuf)

    def _bx(row, off):
        return boxbuf[pl.ds(pl.multiple_of(row * _PER + off, 16), 16)]

    def class_chunk(j, _):
        off = j * 16
        m = scbuf[pl.ds(pl.multiple_of(off, 16), 16)]
        lab = jnp.zeros((16,), jnp.int32)

        def cbody(c, carry):
            mc, lc = carry
            v = scbuf[pl.ds(pl.multiple_of(c * _PER + off, 16), 16)]
            gt = v > mc
            return jnp.where(gt, v, mc), jnp.where(gt, c, lc)

        m, lab = lax.fori_loop(1, num_classes, cbody, (m, lab))
        gidxv = base + off + l16
        smbuf[pl.ds(off, 16)] = jnp.where(gidxv < n_real, m, -1.0)
        lbbuf[pl.ds(off, 16)] = lab
        x1 = _bx(0, off)
        y1 = _bx(1, off)
        x2 = _bx(2, off)
        y2 = _bx(3, off)
        areabuf[pl.ds(off, 16)] = (x2 - x1) * (y2 - y1)
        return 0

    lax.fori_loop(0, _VPT, class_chunk, 0)
    s0buf[0] = smbuf[pl.ds(0, 16)][0]

    def nms_iter(i, _):
        def amax_body(j, carry):
            mc, mic = carry
            off = j * 16
            v = smbuf[pl.ds(off, 16)]
            gt = v > mc
            idxv = base + off + l16
            return jnp.where(gt, v, mc), jnp.where(gt, idxv, mic)

        m0 = jnp.full((16,), -2.0, jnp.float32)
        mi0 = jnp.zeros((16,), jnp.int32)
        m, mi = lax.fori_loop(0, _VPT, amax_body, (m0, mi0))
        bv = m[0]
        bi = mi[0]
        for l in range(1, 16):
            sv = m[l]
            si = mi[l]
            upd = (sv > bv) | ((sv == bv) & (si < bi))
            bv = jnp.where(upd, sv, bv)
            bi = jnp.where(upd, si, bi)

        stagef[...] = jnp.broadcast_to(bv, (16,))
        stagei[...] = jnp.broadcast_to(bi, (16,))
        soff = pl.multiple_of(sid * 16, 16)
        pltpu.sync_copy(stagef, vals_sh.at[pl.ds(soff, 16)])
        pltpu.sync_copy(stagei, idx_sh.at[pl.ds(soff, 16)])
        plsc.subcore_barrier()

        pltpu.sync_copy(vals_sh, valbuf)
        pltpu.sync_copy(idx_sh, idxbuf)
        vbest = valbuf[pl.ds(0, 16)]
        ibest = idxbuf[pl.ds(0, 16)]
        for r in range(1, _NS):
            vr = valbuf[pl.ds(r * 16, 16)]
            ir = idxbuf[pl.ds(r * 16, 16)]
            upd = (vr > vbest) | ((vr == vbest) & (ir < ibest))
            vbest = jnp.where(upd, vr, vbest)
            ibest = jnp.where(upd, ir, ibest)
        gmax = vbest[0]
        gidx = ibest[0]
        any_alive = gmax >= 0.0
        eff = jnp.where(any_alive, gidx, 0)
        wsid = eff // _PER
        li = eff - wsid * _PER
        lic = pl.multiple_of((li // 16) * 16, 16)
        lane = li - lic

        @pl.when(sid == wsid)
        def _publish():
            xv = boxbuf[pl.ds(pl.multiple_of(0 * _PER + lic, 16), 16)]
            yv = boxbuf[pl.ds(pl.multiple_of(1 * _PER + lic, 16), 16)]
            xv2 = boxbuf[pl.ds(pl.multiple_of(2 * _PER + lic, 16), 16)]
            yv2 = boxbuf[pl.ds(pl.multiple_of(3 * _PER + lic, 16), 16)]
            sv = smbuf[pl.ds(lic, 16)]
            lv = lbbuf[pl.ds(lic, 16)]
            bx1 = _dyn_lane(xv, lane)
            by1 = _dyn_lane(yv, lane)
            bx2 = _dyn_lane(xv2, lane)
            by2 = _dyn_lane(yv2, lane)
            sval = _dyn_lane(sv, lane)
            sval = jnp.where(any_alive, sval, s0buf[0])
            lbl = _dyn_lane(lv, lane)
            wf = jnp.where(l16 == 0, bx1,
                 jnp.where(l16 == 1, by1,
                 jnp.where(l16 == 2, bx2,
                 jnp.where(l16 == 3, by2,
                 jnp.where(l16 == 4, sval, 0.0)))))
            stagef[...] = wf
            stagei[...] = jnp.where(l16 == 0, lbl, 0)
            pltpu.sync_copy(stagef, winf_sh)
            pltpu.sync_copy(stagei, wini_sh)
            smbuf[pl.ds(lic, 16)] = jnp.where(l16 == lane, -1.0, sv)

        plsc.subcore_barrier()

        pltpu.sync_copy(winf_sh, winfbuf)
        pltpu.sync_copy(wini_sh, winibuf)
        wv = winfbuf[...]
        wi = winibuf[...]
        bx1 = wv[0]
        by1 = wv[1]
        bx2 = wv[2]
        by2 = wv[3]

        @pl.when(sid == 0)
        def _record():
            ro = pl.multiple_of(i * 16, 16)
            outfb[pl.ds(ro, 16)] = wv
            outib[pl.ds(ro, 16)] = wi

        area_b = (bx2 - bx1) * (by2 - by1)
        thr_eff = jnp.where(any_alive, _IOU_THR, -1.0)

        def iou_body(j, _):
            off = j * 16
            x1v = _bx(0, off)
            y1v = _bx(1, off)
            x2v = _bx(2, off)
            y2v = _bx(3, off)
            smv = smbuf[pl.ds(off, 16)]
            av = areabuf[pl.ds(off, 16)]
            xx1 = jnp.maximum(x1v, bx1)
            yy1 = jnp.maximum(y1v, by1)
            xx2 = jnp.minimum(x2v, bx2)
            yy2 = jnp.minimum(y2v, by2)
            w = jnp.maximum(xx2 - xx1, 0.0)
            h = jnp.maximum(yy2 - yy1, 0.0)
            inter = w * h
            iou = inter / (av + area_b - inter + 1e-9)
            keepv = iou <= thr_eff
            smbuf[pl.ds(off, 16)] = jnp.where(keepv, smv, -1.0)
            return 0

        lax.fori_loop(0, _VPT, iou_body, 0)
        return 0

    lax.fori_loop(0, _MAX_OUT, nms_iter, 0)

    @pl.when(sid == 0)
    def _write_out():
        pltpu.sync_copy(outfb, out_f)
        pltpu.sync_copy(outib, out_i)


def kernel(boxes, scores):
    n = boxes.shape[1]
    num_classes = scores.shape[2]
    boxes_f = boxes.reshape(n, 4)
    scores_f = scores.reshape(n, num_classes)
    pad = _NP - n
    bt = (jnp.pad(boxes_f, ((0, pad), (0, 0)))
          .reshape(_NS, _PER, 4).transpose(0, 2, 1).reshape(-1))
    st = (jnp.pad(scores_f, ((0, pad), (0, 0)))
          .reshape(_NS, _PER, num_classes).transpose(0, 2, 1).reshape(-1))

    mesh = plsc.VectorSubcoreMesh(
        core_axis_name="c", subcore_axis_name="s", num_cores=1)

    sc_nms = pl.kernel(
        functools.partial(_sc_body, n, num_classes),
        out_type=(
            jax.ShapeDtypeStruct((_OUT_PAD * 16,), jnp.float32),
            jax.ShapeDtypeStruct((_OUT_PAD * 16,), jnp.int32),
        ),
        mesh=mesh,
        scratch_types=[
            pltpu.VMEM((num_classes * _PER,), jnp.float32),
            pltpu.VMEM((4 * _PER,), jnp.float32),
            pltpu.VMEM((_PER,), jnp.float32),
            pltpu.VMEM((_PER,), jnp.int32),
            pltpu.VMEM((_PER,), jnp.float32),
            pltpu.VMEM((16,), jnp.float32),
            pltpu.VMEM((16,), jnp.int32),
            pltpu.VMEM((_NS * 16,), jnp.float32),
            pltpu.VMEM((_NS * 16,), jnp.int32),
            pltpu.VMEM((16,), jnp.float32),
            pltpu.VMEM((16,), jnp.int32),
            pltpu.VMEM((_OUT_PAD * 16,), jnp.float32),
            pltpu.VMEM((_OUT_PAD * 16,), jnp.int32),
            pltpu.SMEM((1,), jnp.float32),
            pltpu.VMEM_SHARED((_NS * 16,), jnp.float32),
            pltpu.VMEM_SHARED((_NS * 16,), jnp.int32),
            pltpu.VMEM_SHARED((16,), jnp.float32),
            pltpu.VMEM_SHARED((16,), jnp.int32),
        ],
    )
    out_f, out_i = sc_nms(bt, st)
    wf = out_f.reshape(_OUT_PAD, 16)[:_MAX_OUT]
    wi = out_i.reshape(_OUT_PAD, 16)[:_MAX_OUT]
    return wf[:, :4][None], wf[:, 4][None], wi[:, 0][None]


# speedup vs baseline: 1.1999x; 1.1999x over previous
import functools

import jax
import jax.numpy as jnp
from jax import lax
from jax.experimental import pallas as pl
from jax.experimental.pallas import tpu as pltpu
from jax.experimental.pallas import tpu_sc as plsc

_IOU_THR = 0.6
_MAX_OUT = 100
_NS = 16
_PER = 1280
_NP = _NS * _PER
_VPT = _PER // 16
_OUT_PAD = 112


def _sc_body(n_real, num_classes,
             bt_hbm, st_hbm, out_f, out_i,
             scbuf, boxbuf, smbuf, lbbuf, areabuf,
             stagef, stagei, valbuf, idxbuf, winfbuf, winibuf,
             outfb, outib, s0buf,
             vals_sh, idx_sh, winf_sh, wini_sh):
    sid = lax.axis_index("s")
    base = sid * _PER
    l16 = lax.broadcasted_iota(jnp.int32, (16,), 0)

    nbox = 4 * _PER
    nsc = num_classes * _PER
    pltpu.sync_copy(bt_hbm.at[pl.ds(sid * nbox, nbox)], boxbuf)
    pltpu.sync_copy(st_hbm.at[pl.ds(sid * nsc, nsc)], scbuf)

    def _bx(row, off):
        return boxbuf[pl.ds(pl.multiple_of(row * _PER + off, 16), 16)]

    def class_chunk(j, carry):
        gm, gmi = carry
        off = j * 16
        m = jnp.full((16,), -1.0, jnp.float32)
        lab = jnp.zeros((16,), jnp.int32)

        def cbody(k, carry2):
            mc, lc = carry2
            for cc in range(4):
                c = k * 4 + cc
                v = scbuf[pl.ds(pl.multiple_of(c * _PER + off, 16), 16)]
                gt = v > mc
                mc = jnp.where(gt, v, mc)
                lc = jnp.where(gt, c, lc)
            return mc, lc

        m, lab = lax.fori_loop(0, num_classes // 4, cbody, (m, lab))
        gidxv = base + off + l16
        sm = jnp.where(gidxv < n_real, m, -1.0)
        smbuf[pl.ds(off, 16)] = sm
        lbbuf[pl.ds(off, 16)] = lab
        x1 = _bx(0, off)
        y1 = _bx(1, off)
        x2 = _bx(2, off)
        y2 = _bx(3, off)
        areabuf[pl.ds(off, 16)] = (x2 - x1) * (y2 - y1)
        gt = sm > gm
        return jnp.where(gt, sm, gm), jnp.where(gt, gidxv, gmi)

    gm0 = jnp.full((16,), -2.0, jnp.float32)
    gmi0 = jnp.zeros((16,), jnp.int32)
    gm, gmi = lax.fori_loop(0, _VPT, class_chunk, (gm0, gmi0))
    s0buf[0] = smbuf[pl.ds(0, 16)][0]

    def nms_iter(i, carry):
        gm, gmi = carry
        stagef[...] = gm
        stagei[...] = gmi
        soff = pl.multiple_of(sid * 16, 16)
        pltpu.sync_copy(stagef, vals_sh.at[pl.ds(soff, 16)])
        pltpu.sync_copy(stagei, idx_sh.at[pl.ds(soff, 16)])
        plsc.subcore_barrier()

        pltpu.sync_copy(vals_sh, valbuf)
        pltpu.sync_copy(idx_sh, idxbuf)
        vbest = valbuf[pl.ds(0, 16)]
        ibest = idxbuf[pl.ds(0, 16)]
        for r in range(1, _NS):
            vr = valbuf[pl.ds(r * 16, 16)]
            ir = idxbuf[pl.ds(r * 16, 16)]
            upd = (vr > vbest) | ((vr == vbest) & (ir < ibest))
            vbest = jnp.where(upd, vr, vbest)
            ibest = jnp.where(upd, ir, ibest)
        gmax = vbest[0]
        gidx = ibest[0]
        for l in range(1, 16):
            sv = vbest[l]
            si = ibest[l]
            upd = (sv > gmax) | ((sv == gmax) & (si < gidx))
            gmax = jnp.where(upd, sv, gmax)
            gidx = jnp.where(upd, si, gidx)
        any_alive = gmax >= 0.0
        eff = jnp.where(any_alive, gidx, 0)
        wsid = eff // _PER
        li = eff - wsid * _PER
        lic = pl.multiple_of((li // 16) * 16, 16)
        lane = li - lic

        @pl.when(sid == wsid)
        def _publish():
            xv = boxbuf[pl.ds(pl.multiple_of(0 * _PER + lic, 16), 16)]
            yv = boxbuf[pl.ds(pl.multiple_of(1 * _PER + lic, 16), 16)]
            xv2 = boxbuf[pl.ds(pl.multiple_of(2 * _PER + lic, 16), 16)]
            yv2 = boxbuf[pl.ds(pl.multiple_of(3 * _PER + lic, 16), 16)]
            sv = smbuf[pl.ds(lic, 16)]
            lv = lbbuf[pl.ds(lic, 16)]
            bx1 = xv[0]
            by1 = yv[0]
            bx2 = xv2[0]
            by2 = yv2[0]
            sval = sv[0]
            lbl = lv[0]
            for l in range(1, 16):
                sel = lane == l
                bx1 = jnp.where(sel, xv[l], bx1)
                by1 = jnp.where(sel, yv[l], by1)
                bx2 = jnp.where(sel, xv2[l], bx2)
                by2 = jnp.where(sel, yv2[l], by2)
                sval = jnp.where(sel, sv[l], sval)
                lbl = jnp.where(sel, lv[l], lbl)
            sval = jnp.where(any_alive, sval, s0buf[0])
            wf = jnp.where(l16 == 0, bx1,
                 jnp.where(l16 == 1, by1,
                 jnp.where(l16 == 2, bx2,
                 jnp.where(l16 == 3, by2,
                 jnp.where(l16 == 4, sval, 0.0)))))
            stagef[...] = wf
            stagei[...] = jnp.where(l16 == 0, lbl, 0)
            pltpu.sync_copy(stagef, winf_sh)
            pltpu.sync_copy(stagei, wini_sh)

        plsc.subcore_barrier()

        pltpu.sync_copy(winf_sh, winfbuf)
        pltpu.sync_copy(wini_sh, winibuf)
        wv = winfbuf[...]
        wi = winibuf[...]
        bx1 = wv[0]
        by1 = wv[1]
        bx2 = wv[2]
        by2 = wv[3]

        @pl.when(sid == 0)
        def _record():
            ro = pl.multiple_of(i * 16, 16)
            outfb[pl.ds(ro, 16)] = wv
            outib[pl.ds(ro, 16)] = wi

        area_b = (bx2 - bx1) * (by2 - by1)
        thr_eff = jnp.where(any_alive, _IOU_THR, -1.0)

        def iou_body(j, carry2):
            nm, nmi = carry2
            for cc in range(4):
                off = j * 64 + cc * 16
                x1v = _bx(0, off)
                y1v = _bx(1, off)
                x2v = _bx(2, off)
                y2v = _bx(3, off)
                smv = smbuf[pl.ds(off, 16)]
                av = areabuf[pl.ds(off, 16)]
                xx1 = jnp.maximum(x1v, bx1)
                yy1 = jnp.maximum(y1v, by1)
                xx2 = jnp.minimum(x2v, bx2)
                yy2 = jnp.minimum(y2v, by2)
                w = jnp.maximum(xx2 - xx1, 0.0)
                h = jnp.maximum(yy2 - yy1, 0.0)
                inter = w * h
                iou = inter / (av + area_b - inter + 1e-9)
                smn = jnp.where(iou <= thr_eff, smv, -1.0)
                smbuf[pl.ds(off, 16)] = smn
                gt = smn > nm
                nm = jnp.where(gt, smn, nm)
                nmi = jnp.where(gt, base + off + l16, nmi)
            return nm, nmi

        return lax.fori_loop(0, _VPT // 4, iou_body, (gm0, gmi0))

    lax.fori_loop(0, _MAX_OUT, nms_iter, (gm, gmi))

    @pl.when(sid == 0)
    def _write_out():
        pltpu.sync_copy(outfb, out_f)
        pltpu.sync_copy(outib, out_i)


def kernel(boxes, scores):
    n = boxes.shape[1]
    num_classes = scores.shape[2]
    boxes_f = boxes.reshape(n, 4)
    scores_f = scores.reshape(n, num_classes)
    pad = _NP - n
    bt = (jnp.pad(boxes_f, ((0, pad), (0, 0)))
          .reshape(_NS, _PER, 4).transpose(0, 2, 1).reshape(-1))
    st = (jnp.pad(scores_f, ((0, pad), (0, 0)))
          .reshape(_NS, _PER, num_classes).transpose(0, 2, 1).reshape(-1))

    mesh = plsc.VectorSubcoreMesh(
        core_axis_name="c", subcore_axis_name="s", num_cores=1)

    sc_nms = pl.kernel(
        functools.partial(_sc_body, n, num_classes),
        out_type=(
            jax.ShapeDtypeStruct((_OUT_PAD * 16,), jnp.float32),
            jax.ShapeDtypeStruct((_OUT_PAD * 16,), jnp.int32),
        ),
        mesh=mesh,
        scratch_types=[
            pltpu.VMEM((num_classes * _PER,), jnp.float32),
            pltpu.VMEM((4 * _PER,), jnp.float32),
            pltpu.VMEM((_PER,), jnp.float32),
            pltpu.VMEM((_PER,), jnp.int32),
            pltpu.VMEM((_PER,), jnp.float32),
            pltpu.VMEM((16,), jnp.float32),
            pltpu.VMEM((16,), jnp.int32),
            pltpu.VMEM((_NS * 16,), jnp.float32),
            pltpu.VMEM((_NS * 16,), jnp.int32),
            pltpu.VMEM((16,), jnp.float32),
            pltpu.VMEM((16,), jnp.int32),
            pltpu.VMEM((_OUT_PAD * 16,), jnp.float32),
            pltpu.VMEM((_OUT_PAD * 16,), jnp.int32),
            pltpu.SMEM((1,), jnp.float32),
            pltpu.VMEM_SHARED((_NS * 16,), jnp.float32),
            pltpu.VMEM_SHARED((_NS * 16,), jnp.int32),
            pltpu.VMEM_SHARED((16,), jnp.float32),
            pltpu.VMEM_SHARED((16,), jnp.int32),
        ],
    )
    out_f, out_i = sc_nms(bt, st)
    wf = out_f.reshape(_OUT_PAD, 16)[:_MAX_OUT]
    wi = out_i.reshape(_OUT_PAD, 16)[:_MAX_OUT]
    return wf[:, :4][None], wf[:, 4][None], wi[:, 0][None]


# speedup vs baseline: 1.3392x; 1.1161x over previous
import functools

import jax
import jax.numpy as jnp
from jax import lax
from jax.experimental import pallas as pl
from jax.experimental.pallas import tpu as pltpu
from jax.experimental.pallas import tpu_sc as plsc

_IOU_THR = 0.6
_MAX_OUT = 100
_NS = 16
_PER = 1280
_NP = _NS * _PER
_VPT = _PER // 16
_OUT_PAD = 112


def _sc_body(n_real, num_classes,
             bt_hbm, st_hbm, out_f, out_i,
             scbuf, boxbuf, smbuf, lbbuf, areabuf,
             stagef, stagei, valbuf, idxbuf, outfb, outib, s0buf,
             vals_sh, idx_sh):
    sid = lax.axis_index("s")
    base = sid * _PER
    l16 = lax.broadcasted_iota(jnp.int32, (16,), 0)

    nbox = 4 * _PER
    nsc = num_classes * _PER
    pltpu.sync_copy(bt_hbm.at[pl.ds(sid * nbox, nbox)], boxbuf)
    pltpu.sync_copy(st_hbm.at[pl.ds(sid * nsc, nsc)], scbuf)

    def _bx(row, off):
        return boxbuf[pl.ds(pl.multiple_of(row * _PER + off, 16), 16)]

    def class_chunk(j, carry):
        gm, gmi = carry
        off = j * 16
        m = jnp.full((16,), -1.0, jnp.float32)
        lab = jnp.zeros((16,), jnp.int32)

        def cbody(k, carry2):
            mc, lc = carry2
            for cc in range(4):
                c = k * 4 + cc
                v = scbuf[pl.ds(pl.multiple_of(c * _PER + off, 16), 16)]
                gt = v > mc
                mc = jnp.where(gt, v, mc)
                lc = jnp.where(gt, c, lc)
            return mc, lc

        m, lab = lax.fori_loop(0, num_classes // 4, cbody, (m, lab))
        gidxv = base + off + l16
        sm = jnp.where(gidxv < n_real, m, -1.0)
        smbuf[pl.ds(off, 16)] = sm
        lbbuf[pl.ds(off, 16)] = lab
        x1 = _bx(0, off)
        y1 = _bx(1, off)
        x2 = _bx(2, off)
        y2 = _bx(3, off)
        areabuf[pl.ds(off, 16)] = (x2 - x1) * (y2 - y1)
        gt = sm > gm
        return jnp.where(gt, sm, gm), jnp.where(gt, gidxv, gmi)

    gm0 = jnp.full((16,), -2.0, jnp.float32)
    gmi0 = jnp.zeros((16,), jnp.int32)
    gm, gmi = lax.fori_loop(0, _VPT, class_chunk, (gm0, gmi0))
    s0buf[0] = smbuf[pl.ds(0, 16)][0]

    def nms_iter(i, carry):
        gm, gmi = carry
        bv = gm[0]
        bi = gmi[0]
        for l in range(1, 16):
            sv = gm[l]
            si = gmi[l]
            upd = (sv > bv) | ((sv == bv) & (si < bi))
            bv = jnp.where(upd, sv, bv)
            bi = jnp.where(upd, si, bi)
        lli = bi - base
        lic = pl.multiple_of((lli // 16) * 16, 16)
        lane = lli - lic
        xv = boxbuf[pl.ds(pl.multiple_of(0 * _PER + lic, 16), 16)]
        yv = boxbuf[pl.ds(pl.multiple_of(1 * _PER + lic, 16), 16)]
        xv2 = boxbuf[pl.ds(pl.multiple_of(2 * _PER + lic, 16), 16)]
        yv2 = boxbuf[pl.ds(pl.multiple_of(3 * _PER + lic, 16), 16)]
        sv = smbuf[pl.ds(lic, 16)]
        lv = lbbuf[pl.ds(lic, 16)]
        bx1 = xv[0]
        by1 = yv[0]
        bx2 = xv2[0]
        by2 = yv2[0]
        sval = sv[0]
        lbl = lv[0]
        for l in range(1, 16):
            sel = lane == l
            bx1 = jnp.where(sel, xv[l], bx1)
            by1 = jnp.where(sel, yv[l], by1)
            bx2 = jnp.where(sel, xv2[l], bx2)
            by2 = jnp.where(sel, yv2[l], by2)
            sval = jnp.where(sel, sv[l], sval)
            lbl = jnp.where(sel, lv[l], lbl)
        rowf = jnp.where(l16 == 0, bx1,
               jnp.where(l16 == 1, by1,
               jnp.where(l16 == 2, bx2,
               jnp.where(l16 == 3, by2,
               jnp.where(l16 == 4, sval, bv)))))
        rowi = jnp.where(l16 == 0, lbl, jnp.broadcast_to(bi, (16,)))
        stagef[...] = rowf
        stagei[...] = rowi
        po = pl.multiple_of((i % 2) * (_NS * 16), 16)
        soff = pl.multiple_of(po + sid * 16, 16)
        pltpu.sync_copy(stagef, vals_sh.at[pl.ds(soff, 16)])
        pltpu.sync_copy(stagei, idx_sh.at[pl.ds(soff, 16)])
        plsc.subcore_barrier()
        pltpu.sync_copy(vals_sh.at[pl.ds(po, _NS * 16)], valbuf)
        pltpu.sync_copy(idx_sh.at[pl.ds(po, _NS * 16)], idxbuf)

        r0 = valbuf[pl.ds(0, 16)]
        i0 = idxbuf[pl.ds(0, 16)]
        bmax = r0[5]
        bidx = i0[1]
        bt_row = 0
        for r in range(1, _NS):
            rv = valbuf[pl.ds(r * 16, 16)][5]
            ri = idxbuf[pl.ds(r * 16, 16)][1]
            upd = (rv > bmax) | ((rv == bmax) & (ri < bidx))
            bmax = jnp.where(upd, rv, bmax)
            bidx = jnp.where(upd, ri, bidx)
            bt_row = jnp.where(upd, r, bt_row)
        any_alive = bmax >= 0.0
        wro = pl.multiple_of(bt_row * 16, 16)
        wrow_f = valbuf[pl.ds(wro, 16)]
        wrow_i = idxbuf[pl.ds(wro, 16)]
        bx1 = wrow_f[0]
        by1 = wrow_f[1]
        bx2 = wrow_f[2]
        by2 = wrow_f[3]

        @pl.when(sid == 0)
        def _record():
            x0 = _bx(0, 0)[0]
            y0 = _bx(1, 0)[0]
            x20 = _bx(2, 0)[0]
            y20 = _bx(3, 0)[0]
            lb0 = lbbuf[pl.ds(0, 16)][0]
            rb1 = jnp.where(any_alive, bx1, x0)
            rb2 = jnp.where(any_alive, by1, y0)
            rb3 = jnp.where(any_alive, bx2, x20)
            rb4 = jnp.where(any_alive, by2, y20)
            rsc = jnp.where(any_alive, wrow_f[4], s0buf[0])
            rlb = jnp.where(any_alive, wrow_i[0], lb0)
            recf = jnp.where(l16 == 0, rb1,
                   jnp.where(l16 == 1, rb2,
                   jnp.where(l16 == 2, rb3,
                   jnp.where(l16 == 3, rb4, rsc))))
            reci = jnp.where(l16 == 0, rlb, 0)
            ro = pl.multiple_of(i * 16, 16)
            outfb[pl.ds(ro, 16)] = recf
            outib[pl.ds(ro, 16)] = reci

        area_b = (bx2 - bx1) * (by2 - by1)
        thr_eff = jnp.where(any_alive, _IOU_THR, -1.0)

        def iou_body(j, carry2):
            nm, nmi = carry2
            for cc in range(4):
                off = j * 64 + cc * 16
                x1v = _bx(0, off)
                y1v = _bx(1, off)
                x2v = _bx(2, off)
                y2v = _bx(3, off)
                smv = smbuf[pl.ds(off, 16)]
                av = areabuf[pl.ds(off, 16)]
                xx1 = jnp.maximum(x1v, bx1)
                yy1 = jnp.maximum(y1v, by1)
                xx2 = jnp.minimum(x2v, bx2)
                yy2 = jnp.minimum(y2v, by2)
                w = jnp.maximum(xx2 - xx1, 0.0)
                h = jnp.maximum(yy2 - yy1, 0.0)
                inter = w * h
                iou = inter / (av + area_b - inter + 1e-9)
                smn = jnp.where(iou <= thr_eff, smv, -1.0)
                smbuf[pl.ds(off, 16)] = smn
                gt = smn > nm
                nm = jnp.where(gt, smn, nm)
                nmi = jnp.where(gt, base + off + l16, nmi)
            return nm, nmi

        return lax.fori_loop(0, _VPT // 4, iou_body, (gm0, gmi0))

    lax.fori_loop(0, _MAX_OUT, nms_iter, (gm, gmi))

    @pl.when(sid == 0)
    def _write_out():
        pltpu.sync_copy(outfb, out_f)
        pltpu.sync_copy(outib, out_i)


def kernel(boxes, scores):
    n = boxes.shape[1]
    num_classes = scores.shape[2]
    boxes_f = boxes.reshape(n, 4)
    scores_f = scores.reshape(n, num_classes)
    pad = _NP - n
    bt = (jnp.pad(boxes_f, ((0, pad), (0, 0)))
          .reshape(_NS, _PER, 4).transpose(0, 2, 1).reshape(-1))
    st = (jnp.pad(scores_f, ((0, pad), (0, 0)))
          .reshape(_NS, _PER, num_classes).transpose(0, 2, 1).reshape(-1))

    mesh = plsc.VectorSubcoreMesh(
        core_axis_name="c", subcore_axis_name="s", num_cores=1)

    sc_nms = pl.kernel(
        functools.partial(_sc_body, n, num_classes),
        out_type=(
            jax.ShapeDtypeStruct((_OUT_PAD * 16,), jnp.float32),
            jax.ShapeDtypeStruct((_OUT_PAD * 16,), jnp.int32),
        ),
        mesh=mesh,
        scratch_types=[
            pltpu.VMEM((num_classes * _PER,), jnp.float32),
            pltpu.VMEM((4 * _PER,), jnp.float32),
            pltpu.VMEM((_PER,), jnp.float32),
            pltpu.VMEM((_PER,), jnp.int32),
            pltpu.VMEM((_PER,), jnp.float32),
            pltpu.VMEM((16,), jnp.float32),
            pltpu.VMEM((16,), jnp.int32),
            pltpu.VMEM((_NS * 16,), jnp.float32),
            pltpu.VMEM((_NS * 16,), jnp.int32),
            pltpu.VMEM((_OUT_PAD * 16,), jnp.float32),
            pltpu.VMEM((_OUT_PAD * 16,), jnp.int32),
            pltpu.SMEM((1,), jnp.float32),
            pltpu.VMEM_SHARED((2 * _NS * 16,), jnp.float32),
            pltpu.VMEM_SHARED((2 * _NS * 16,), jnp.int32),
        ],
    )
    out_f, out_i = sc_nms(bt, st)
    wf = out_f.reshape(_OUT_PAD, 16)[:_MAX_OUT]
    wi = out_i.reshape(_OUT_PAD, 16)[:_MAX_OUT]
    return wf[:, :4][None], wf[:, 4][None], wi[:, 0][None]


# speedup vs baseline: 1.4550x; 1.0865x over previous
import functools

import jax
import jax.numpy as jnp
from jax import lax
from jax.experimental import pallas as pl
from jax.experimental.pallas import tpu as pltpu
from jax.experimental.pallas import tpu_sc as plsc

_IOU_THR = 0.6
_MAX_OUT = 100
_NS = 16
_PER = 1280
_NP = _NS * _PER
_VPT = _PER // 16
_OUT_PAD = 112


def _sc_body(n_real, num_classes,
             bt_hbm, st_hbm, out_f, out_i,
             scbuf, boxbuf, smbuf, lbbuf,
             stagef, stagei, valbuf, idxbuf, outfb, outib, s0buf,
             sem1, sem2, vals_sh, idx_sh):
    sid = lax.axis_index("s")
    base = sid * _PER
    l16 = lax.broadcasted_iota(jnp.int32, (16,), 0)

    nbox = 4 * _PER
    nsc = num_classes * _PER
    pltpu.sync_copy(bt_hbm.at[pl.ds(sid * nbox, nbox)],
                    boxbuf.at[pl.ds(0, nbox)])
    pltpu.sync_copy(st_hbm.at[pl.ds(sid * nsc, nsc)], scbuf)

    def _bx(row, off):
        return boxbuf[pl.ds(pl.multiple_of(row * _PER + off, 16), 16)]

    def class_chunk(j, carry):
        gm, gmi = carry
        off = j * 16
        m = jnp.full((16,), -1.0, jnp.float32)
        lab = jnp.zeros((16,), jnp.int32)

        def cbody(k, carry2):
            mc, lc = carry2
            for cc in range(8):
                c = k * 8 + cc
                v = scbuf[pl.ds(pl.multiple_of(c * _PER + off, 16), 16)]
                gt = v > mc
                mc = jnp.where(gt, v, mc)
                lc = jnp.where(gt, c, lc)
            return mc, lc

        m, lab = lax.fori_loop(0, num_classes // 8, cbody, (m, lab))
        gidxv = base + off + l16
        sm = jnp.where(gidxv < n_real, m, -1.0)
        smbuf[pl.ds(off, 16)] = sm
        lbbuf[pl.ds(off, 16)] = lab
        gt = sm > gm
        return jnp.where(gt, sm, gm), jnp.where(gt, gidxv, gmi)

    gm0 = jnp.full((16,), -2.0, jnp.float32)
    gmi0 = jnp.zeros((16,), jnp.int32)
    gm, gmi = lax.fori_loop(0, _VPT, class_chunk, (gm0, gmi0))
    s0buf[0] = smbuf[pl.ds(0, 16)][0]

    def nms_iter(i, carry):
        gm, gmi = carry
        bv = gm[0]
        bi = gmi[0]
        for l in range(1, 16):
            sv = gm[l]
            si = gmi[l]
            upd = (sv > bv) | ((sv == bv) & (si < bi))
            bv = jnp.where(upd, sv, bv)
            bi = jnp.where(upd, si, bi)
        lli = bi - base
        bx1 = boxbuf[pl.ds(0 * _PER + lli, 16)][0]
        by1 = boxbuf[pl.ds(1 * _PER + lli, 16)][0]
        bx2 = boxbuf[pl.ds(2 * _PER + lli, 16)][0]
        by2 = boxbuf[pl.ds(3 * _PER + lli, 16)][0]
        sval = smbuf[pl.ds(lli, 16)][0]
        lbl = lbbuf[pl.ds(lli, 16)][0]
        rowf = jnp.where(l16 == 0, bx1,
               jnp.where(l16 == 1, by1,
               jnp.where(l16 == 2, bx2,
               jnp.where(l16 == 3, by2,
               jnp.where(l16 == 4, sval, bv)))))
        rowi = jnp.where(l16 == 0, lbl, jnp.broadcast_to(bi, (16,)))
        stagef[...] = rowf
        stagei[...] = rowi
        po = pl.multiple_of((i % 2) * (_NS * 16), 16)
        soff = pl.multiple_of(po + sid * 16, 16)
        w1 = pltpu.async_copy(stagef, vals_sh.at[pl.ds(soff, 16)], sem1)
        w2 = pltpu.async_copy(stagei, idx_sh.at[pl.ds(soff, 16)], sem2)
        w1.wait()
        w2.wait()
        plsc.subcore_barrier()
        r1 = pltpu.async_copy(vals_sh.at[pl.ds(po, _NS * 16)], valbuf, sem1)
        r2 = pltpu.async_copy(idx_sh.at[pl.ds(po, _NS * 16)], idxbuf, sem2)
        r1.wait()
        r2.wait()

        r0 = valbuf[pl.ds(0, 16)]
        i0 = idxbuf[pl.ds(0, 16)]
        bmax = r0[5]
        bidx = i0[1]
        bt_row = 0
        for r in range(1, _NS):
            rv = valbuf[pl.ds(r * 16, 16)][5]
            ri = idxbuf[pl.ds(r * 16, 16)][1]
            upd = (rv > bmax) | ((rv == bmax) & (ri < bidx))
            bmax = jnp.where(upd, rv, bmax)
            bidx = jnp.where(upd, ri, bidx)
            bt_row = jnp.where(upd, r, bt_row)
        any_alive = bmax >= 0.0
        wro = pl.multiple_of(bt_row * 16, 16)
        wrow_f = valbuf[pl.ds(wro, 16)]
        wrow_i = idxbuf[pl.ds(wro, 16)]
        bx1 = wrow_f[0]
        by1 = wrow_f[1]
        bx2 = wrow_f[2]
        by2 = wrow_f[3]

        @pl.when(sid == 0)
        def _record():
            x0 = _bx(0, 0)[0]
            y0 = _bx(1, 0)[0]
            x20 = _bx(2, 0)[0]
            y20 = _bx(3, 0)[0]
            lb0 = lbbuf[pl.ds(0, 16)][0]
            rb1 = jnp.where(any_alive, bx1, x0)
            rb2 = jnp.where(any_alive, by1, y0)
            rb3 = jnp.where(any_alive, bx2, x20)
            rb4 = jnp.where(any_alive, by2, y20)
            rsc = jnp.where(any_alive, wrow_f[4], s0buf[0])
            rlb = jnp.where(any_alive, wrow_i[0], lb0)
            recf = jnp.where(l16 == 0, rb1,
                   jnp.where(l16 == 1, rb2,
                   jnp.where(l16 == 2, rb3,
                   jnp.where(l16 == 3, rb4, rsc))))
            reci = jnp.where(l16 == 0, rlb, 0)
            ro = pl.multiple_of(i * 16, 16)
            outfb[pl.ds(ro, 16)] = recf
            outib[pl.ds(ro, 16)] = reci

        area_b = (bx2 - bx1) * (by2 - by1)
        thr_eff = jnp.where(any_alive, _IOU_THR, -1.0)

        def iou_body(j, carry2):
            nm, nmi = carry2
            for cc in range(8):
                off = j * 128 + cc * 16
                x1v = _bx(0, off)
                y1v = _bx(1, off)
                x2v = _bx(2, off)
                y2v = _bx(3, off)
                smv = smbuf[pl.ds(off, 16)]
                av = (x2v - x1v) * (y2v - y1v)
                xx1 = jnp.maximum(x1v, bx1)
                yy1 = jnp.maximum(y1v, by1)
                xx2 = jnp.minimum(x2v, bx2)
                yy2 = jnp.minimum(y2v, by2)
                w = jnp.maximum(xx2 - xx1, 0.0)
                h = jnp.maximum(yy2 - yy1, 0.0)
                inter = w * h
                iou = inter / (av + area_b - inter + 1e-9)
                smn = jnp.where(iou <= thr_eff, smv, -1.0)
                smbuf[pl.ds(off, 16)] = smn
                gt = smn > nm
                nm = jnp.where(gt, smn, nm)
                nmi = jnp.where(gt, base + off + l16, nmi)
            return nm, nmi

        return lax.fori_loop(0, _VPT // 8, iou_body, (gm0, gmi0))

    lax.fori_loop(0, _MAX_OUT, nms_iter, (gm, gmi))

    @pl.when(sid == 0)
    def _write_out():
        pltpu.sync_copy(outfb, out_f)
        pltpu.sync_copy(outib, out_i)


def kernel(boxes, scores):
    n = boxes.shape[1]
    num_classes = scores.shape[2]
    boxes_f = boxes.reshape(n, 4)
    scores_f = scores.reshape(n, num_classes)
    pad = _NP - n
    bt = (jnp.pad(boxes_f, ((0, pad), (0, 0)))
          .reshape(_NS, _PER, 4).transpose(0, 2, 1).reshape(-1))
    st = (jnp.pad(scores_f, ((0, pad), (0, 0)))
          .reshape(_NS, _PER, num_classes).transpose(0, 2, 1).reshape(-1))

    mesh = plsc.VectorSubcoreMesh(
        core_axis_name="c", subcore_axis_name="s", num_cores=1)

    sc_nms = pl.kernel(
        functools.partial(_sc_body, n, num_classes),
        out_type=(
            jax.ShapeDtypeStruct((_OUT_PAD * 16,), jnp.float32),
            jax.ShapeDtypeStruct((_OUT_PAD * 16,), jnp.int32),
        ),
        mesh=mesh,
        scratch_types=[
            pltpu.VMEM((num_classes * _PER,), jnp.float32),
            pltpu.VMEM((4 * _PER + 16,), jnp.float32),
            pltpu.VMEM((_PER + 16,), jnp.float32),
            pltpu.VMEM((_PER + 16,), jnp.int32),
            pltpu.VMEM((16,), jnp.float32),
            pltpu.VMEM((16,), jnp.int32),
            pltpu.VMEM((_NS * 16,), jnp.float32),
            pltpu.VMEM((_NS * 16,), jnp.int32),
            pltpu.VMEM((_OUT_PAD * 16,), jnp.float32),
            pltpu.VMEM((_OUT_PAD * 16,), jnp.int32),
            pltpu.SMEM((1,), jnp.float32),
            pltpu.SemaphoreType.DMA,
            pltpu.SemaphoreType.DMA,
            pltpu.VMEM_SHARED((2 * _NS * 16,), jnp.float32),
            pltpu.VMEM_SHARED((2 * _NS * 16,), jnp.int32),
        ],
    )
    out_f, out_i = sc_nms(bt, st)
    wf = out_f.reshape(_OUT_PAD, 16)[:_MAX_OUT]
    wi = out_i.reshape(_OUT_PAD, 16)[:_MAX_OUT]
    return wf[:, :4][None], wf[:, 4][None], wi[:, 0][None]


# speedup vs baseline: 1.4563x; 1.0008x over previous
import functools

import jax
import jax.numpy as jnp
from jax import lax
from jax.experimental import pallas as pl
from jax.experimental.pallas import tpu as pltpu
from jax.experimental.pallas import tpu_sc as plsc

_IOU_THR = 0.6
_MAX_OUT = 100
_NS = 16
_PER = 1280
_NP = _NS * _PER
_VPT = _PER // 16
_OUT_PAD = 112


def _sc_body(n_real, num_classes,
             bt_hbm, st_hbm, out_f, out_i,
             scbuf, boxbuf, smbuf, lbbuf,
             stagef, valbuf, outfb, outib, s0buf,
             vals_sh):
    sid = lax.axis_index("s")
    base = sid * _PER
    l16 = lax.broadcasted_iota(jnp.int32, (16,), 0)

    nbox = 4 * _PER
    nsc = num_classes * _PER
    pltpu.sync_copy(bt_hbm.at[pl.ds(sid * nbox, nbox)],
                    boxbuf.at[pl.ds(0, nbox)])
    pltpu.sync_copy(st_hbm.at[pl.ds(sid * nsc, nsc)], scbuf)

    def _bx(row, off):
        return boxbuf[pl.ds(pl.multiple_of(row * _PER + off, 16), 16)]

    def class_chunk(j, carry):
        gm, gmi = carry
        off = j * 16
        m = jnp.full((16,), -1.0, jnp.float32)
        lab = jnp.zeros((16,), jnp.int32)

        def cbody(k, carry2):
            mc, lc = carry2
            for cc in range(8):
                c = k * 8 + cc
                v = scbuf[pl.ds(pl.multiple_of(c * _PER + off, 16), 16)]
                gt = v > mc
                mc = jnp.where(gt, v, mc)
                lc = jnp.where(gt, c, lc)
            return mc, lc

        m, lab = lax.fori_loop(0, num_classes // 8, cbody, (m, lab))
        gidxv = base + off + l16
        sm = jnp.where(gidxv < n_real, m, -1.0)
        smbuf[pl.ds(off, 16)] = sm
        lbbuf[pl.ds(off, 16)] = lab
        gt = sm > gm
        return jnp.where(gt, sm, gm), jnp.where(gt, gidxv, gmi)

    gm0 = jnp.full((16,), -2.0, jnp.float32)
    gmi0 = jnp.zeros((16,), jnp.int32)
    gm, gmi = lax.fori_loop(0, _VPT, class_chunk, (gm0, gmi0))
    s0buf[0] = smbuf[pl.ds(0, 16)][0]

    def nms_iter(i, carry):
        gm, gmi = carry
        bv = gm[0]
        bi = gmi[0]
        for l in range(1, 16):
            sv = gm[l]
            si = gmi[l]
            upd = (sv > bv) | ((sv == bv) & (si < bi))
            bv = jnp.where(upd, sv, bv)
            bi = jnp.where(upd, si, bi)
        lli = bi - base
        bx1 = boxbuf[pl.ds(0 * _PER + lli, 16)][0]
        by1 = boxbuf[pl.ds(1 * _PER + lli, 16)][0]
        bx2 = boxbuf[pl.ds(2 * _PER + lli, 16)][0]
        by2 = boxbuf[pl.ds(3 * _PER + lli, 16)][0]
        sval = smbuf[pl.ds(lli, 16)][0]
        lbl = lbbuf[pl.ds(lli, 16)][0]
        rowf = jnp.where(l16 == 0, bx1,
               jnp.where(l16 == 1, by1,
               jnp.where(l16 == 2, bx2,
               jnp.where(l16 == 3, by2,
               jnp.where(l16 == 4, sval,
               jnp.where(l16 == 6, lbl.astype(jnp.float32),
               jnp.where(l16 == 7, bi.astype(jnp.float32), bv)))))))
        stagef[...] = rowf
        po = pl.multiple_of((i % 2) * (_NS * 16), 16)
        soff = pl.multiple_of(po + sid * 16, 16)
        pltpu.sync_copy(stagef, vals_sh.at[pl.ds(soff, 16)])
        plsc.subcore_barrier()
        pltpu.sync_copy(vals_sh.at[pl.ds(po, _NS * 16)], valbuf)

        r0 = valbuf[pl.ds(0, 16)]
        bmax = r0[5]
        bidx = r0[7]
        bt_row = 0
        for r in range(1, _NS):
            rr = valbuf[pl.ds(r * 16, 16)]
            rv = rr[5]
            ri = rr[7]
            upd = (rv > bmax) | ((rv == bmax) & (ri < bidx))
            bmax = jnp.where(upd, rv, bmax)
            bidx = jnp.where(upd, ri, bidx)
            bt_row = jnp.where(upd, r, bt_row)
        any_alive = bmax >= 0.0
        wro = pl.multiple_of(bt_row * 16, 16)
        wrow_f = valbuf[pl.ds(wro, 16)]
        bx1 = wrow_f[0]
        by1 = wrow_f[1]
        bx2 = wrow_f[2]
        by2 = wrow_f[3]

        @pl.when(sid == 0)
        def _record():
            x0 = _bx(0, 0)[0]
            y0 = _bx(1, 0)[0]
            x20 = _bx(2, 0)[0]
            y20 = _bx(3, 0)[0]
            lb0 = lbbuf[pl.ds(0, 16)][0]
            rb1 = jnp.where(any_alive, bx1, x0)
            rb2 = jnp.where(any_alive, by1, y0)
            rb3 = jnp.where(any_alive, bx2, x20)
            rb4 = jnp.where(any_alive, by2, y20)
            rsc = jnp.where(any_alive, wrow_f[4], s0buf[0])
            rlb = jnp.where(any_alive, wrow_f[6].astype(jnp.int32), lb0)
            recf = jnp.where(l16 == 0, rb1,
                   jnp.where(l16 == 1, rb2,
                   jnp.where(l16 == 2, rb3,
                   jnp.where(l16 == 3, rb4, rsc))))
            reci = jnp.where(l16 == 0, rlb, 0)
            ro = pl.multiple_of(i * 16, 16)
            outfb[pl.ds(ro, 16)] = recf
            outib[pl.ds(ro, 16)] = reci

        area_b = (bx2 - bx1) * (by2 - by1)
        thr_eff = jnp.where(any_alive, _IOU_THR, -1.0)

        def iou_body(j, carry2):
            nm, nmi = carry2
            for cc in range(8):
                off = j * 128 + cc * 16
                x1v = _bx(0, off)
                y1v = _bx(1, off)
                x2v = _bx(2, off)
                y2v = _bx(3, off)
                smv = smbuf[pl.ds(off, 16)]
                av = (x2v - x1v) * (y2v - y1v)
                xx1 = jnp.maximum(x1v, bx1)
                yy1 = jnp.maximum(y1v, by1)
                xx2 = jnp.minimum(x2v, bx2)
                yy2 = jnp.minimum(y2v, by2)
                w = jnp.maximum(xx2 - xx1, 0.0)
                h = jnp.maximum(yy2 - yy1, 0.0)
                inter = w * h
                iou = inter / (av + area_b - inter + 1e-9)
                smn = jnp.where(iou <= thr_eff, smv, -1.0)
                smbuf[pl.ds(off, 16)] = smn
                gt = smn > nm
                nm = jnp.where(gt, smn, nm)
                nmi = jnp.where(gt, base + off + l16, nmi)
            return nm, nmi

        return lax.fori_loop(0, _VPT // 8, iou_body, (gm0, gmi0))

    lax.fori_loop(0, _MAX_OUT, nms_iter, (gm, gmi))

    @pl.when(sid == 0)
    def _write_out():
        pltpu.sync_copy(outfb, out_f)
        pltpu.sync_copy(outib, out_i)


def kernel(boxes, scores):
    n = boxes.shape[1]
    num_classes = scores.shape[2]
    boxes_f = boxes.reshape(n, 4)
    scores_f = scores.reshape(n, num_classes)
    pad = _NP - n
    bt = (jnp.pad(boxes_f, ((0, pad), (0, 0)))
          .reshape(_NS, _PER, 4).transpose(0, 2, 1).reshape(-1))
    st = (jnp.pad(scores_f, ((0, pad), (0, 0)))
          .reshape(_NS, _PER, num_classes).transpose(0, 2, 1).reshape(-1))

    mesh = plsc.VectorSubcoreMesh(
        core_axis_name="c", subcore_axis_name="s", num_cores=1)

    sc_nms = pl.kernel(
        functools.partial(_sc_body, n, num_classes),
        out_type=(
            jax.ShapeDtypeStruct((_OUT_PAD * 16,), jnp.float32),
            jax.ShapeDtypeStruct((_OUT_PAD * 16,), jnp.int32),
        ),
        mesh=mesh,
        scratch_types=[
            pltpu.VMEM((num_classes * _PER,), jnp.float32),
            pltpu.VMEM((4 * _PER + 16,), jnp.float32),
            pltpu.VMEM((_PER + 16,), jnp.float32),
            pltpu.VMEM((_PER + 16,), jnp.int32),
            pltpu.VMEM((16,), jnp.float32),
            pltpu.VMEM((_NS * 16,), jnp.float32),
            pltpu.VMEM((_OUT_PAD * 16,), jnp.float32),
            pltpu.VMEM((_OUT_PAD * 16,), jnp.int32),
            pltpu.SMEM((1,), jnp.float32),
            pltpu.VMEM_SHARED((2 * _NS * 16,), jnp.float32),
        ],
    )
    out_f, out_i = sc_nms(bt, st)
    wf = out_f.reshape(_OUT_PAD, 16)[:_MAX_OUT]
    wi = out_i.reshape(_OUT_PAD, 16)[:_MAX_OUT]
    return wf[:, :4][None], wf[:, 4][None], wi[:, 0][None]
